# baseline (speedup 1.0000x reference)
import functools

import jax
import jax.numpy as jnp
from jax import lax
from jax.experimental import pallas as pl
from jax.experimental.pallas import tpu as pltpu
from jax.experimental.pallas import tpu_sc as plsc

_N_SLICES = 4


def _argmin_body(x_ref, cb_ref, c2_ref, x2_ref, idx_ref):
    n = cb_ref.shape[0]
    cb = cb_ref[...]
    c2 = c2_ref[...]
    r = x_ref.shape[0]
    sub = 256
    for s in range(r // sub):
        rows = pl.ds(s * sub, sub)
        x = x_ref[rows, :]
        xc = lax.dot_general(x, cb, (((1,), (1,)), ((), ())),
                             preferred_element_type=jnp.float32)
        dists = (x2_ref[rows, :] + c2) - 2.0 * xc
        mn = jnp.min(dists, axis=1, keepdims=True)
        cand = jax.lax.broadcasted_iota(jnp.int32, dists.shape, 1)
        idx = jnp.min(jnp.where(dists == mn, cand, jnp.int32(n)), axis=1)
        idx_ref[0, 0, rows] = idx


def _nearest_idx(x_flat, cb, c2, x2, block_rows):
    m, d = x_flat.shape
    n = cb.shape[0]
    grid = m // block_rows
    out = pl.pallas_call(
        _argmin_body,
        grid=(grid,),
        in_specs=[
            pl.BlockSpec((block_rows, d), lambda i: (i, 0)),
            pl.BlockSpec((n, d), lambda i: (0, 0)),
            pl.BlockSpec((1, n), lambda i: (0, 0)),
            pl.BlockSpec((block_rows, 1), lambda i: (i, 0)),
        ],
        out_specs=pl.BlockSpec((1, 1, block_rows), lambda i: (i, 0, 0)),
        out_shape=jax.ShapeDtypeStruct((grid, 1, block_rows), jnp.int32),
    )(x_flat, cb, c2.reshape(1, n), x2.reshape(m, 1))
    return out.reshape(m)


def _make_gather_slice(d, b_slice, slice_base):
    info = plsc.get_sparse_core_info()
    nw = info.num_cores * info.num_subcores
    b_per_w = b_slice // nw
    chunk = min(32, b_per_w)
    n_chunks = b_per_w // chunk
    mesh = plsc.VectorSubcoreMesh(core_axis_name="c", subcore_axis_name="s")

    @functools.partial(
        pl.kernel,
        mesh=mesh,
        out_type=(),
        scratch_types=[
            pltpu.VMEM((b_per_w,), jnp.int32),
            pltpu.VMEM((2, chunk, d), jnp.float32),
            pltpu.SemaphoreType.DMA,
            pltpu.SemaphoreType.DMA,
            pltpu.SemaphoreType.DMA,
        ],
    )
    def gather(table_hbm, idx_hbm, out_hbm, idx_v, rows_v, gsem, wsem0, wsem1):
        wid = lax.axis_index("s") * info.num_cores + lax.axis_index("c")
        base = wid * b_per_w
        wsems = (wsem0, wsem1)
        pltpu.sync_copy(idx_hbm.at[pl.ds(base, b_per_w)], idx_v)
        writebacks = [None, None]
        for c in range(n_chunks):
            sl = c % 2
            g = pltpu.make_async_copy(
                table_hbm.at[idx_v.at[pl.ds(c * chunk, chunk)]],
                rows_v.at[sl], gsem)
            g.start()
            if writebacks[sl] is not None:
                writebacks[sl].wait()
            g.wait()
            w = pltpu.make_async_copy(
                rows_v.at[sl],
                out_hbm.at[pl.ds(slice_base + base + c * chunk, chunk)],
                wsems[sl])
            w.start()
            writebacks[sl] = w
        writebacks[0].wait()
        if writebacks[1] is not None:
            writebacks[1].wait()

    return gather


def kernel(x, codebook):
    b, t, d = x.shape
    n = codebook.shape[0]
    m = b * t
    x_flat = x.reshape(m, d)
    x2 = (x_flat ** 2).sum(axis=1)
    c2 = (codebook ** 2).sum(axis=1)
    m_s = m // _N_SLICES
    out_ref = jax.new_ref(jnp.zeros((m, d), jnp.float32))
    for k in range(_N_SLICES):
        sl = pl.ds(k * m_s, m_s)
        idx_k = _nearest_idx(x_flat[sl], codebook, c2, x2[sl], block_rows=1024)
        _make_gather_slice(d, m_s, k * m_s)(codebook, idx_k, out_ref)
    return out_ref[...].reshape(b, t, d)


# speedup vs baseline: 1.7296x; 1.7296x over previous
import functools

import jax
import jax.numpy as jnp
from jax import lax
from jax.experimental import pallas as pl
from jax.experimental.pallas import tpu as pltpu
from jax.experimental.pallas import tpu_sc as plsc


def _argmin_body(x_ref, cb_ref, c2_ref, idx_ref):
    n = cb_ref.shape[0]
    cb = cb_ref[...]
    c2 = c2_ref[...]
    r = x_ref.shape[0]
    sub = 256
    for s in range(r // sub):
        rows = pl.ds(s * sub, sub)
        x = x_ref[rows, :]
        xc = lax.dot_general(x, cb, (((1,), (1,)), ((), ())),
                             preferred_element_type=jnp.float32)
        x2 = jnp.sum(x * x, axis=1, keepdims=True)
        dists = (x2 + c2) - 2.0 * xc
        mn = jnp.min(dists, axis=1, keepdims=True)
        cand = jax.lax.broadcasted_iota(jnp.int32, dists.shape, 1)
        idx = jnp.min(jnp.where(dists == mn, cand, jnp.int32(n)), axis=1)
        idx_ref[0, 0, rows] = idx


def _nearest_idx(x_flat, cb, c2, block_rows):
    m, d = x_flat.shape
    n = cb.shape[0]
    n_blocks = m // block_rows
    out = pl.pallas_call(
        _argmin_body,
        grid=(n_blocks,),
        in_specs=[
            pl.BlockSpec((block_rows, d), lambda i: (i, 0)),
            pl.BlockSpec((n, d), lambda i: (0, 0)),
            pl.BlockSpec((1, n), lambda i: (0, 0)),
        ],
        out_specs=pl.BlockSpec((1, 1, block_rows), lambda i: (i, 0, 0)),
        out_shape=jax.ShapeDtypeStruct((n_blocks, 1, block_rows), jnp.int32),
    )(x_flat, cb, c2.reshape(1, n))
    return out.reshape(m)


def _make_gather(d, b_total):
    info = plsc.get_sparse_core_info()
    nw = info.num_cores * info.num_subcores
    b_per_w = b_total // nw
    chunk = min(32, b_per_w)
    n_chunks = b_per_w // chunk
    mesh = plsc.VectorSubcoreMesh(core_axis_name="c", subcore_axis_name="s")

    @functools.partial(
        pl.kernel,
        mesh=mesh,
        out_type=jax.ShapeDtypeStruct((b_total, d), jnp.float32),
        scratch_types=[
            pltpu.VMEM((b_per_w,), jnp.int32),
            pltpu.VMEM((2, chunk, d), jnp.float32),
            pltpu.SemaphoreType.DMA,
            pltpu.SemaphoreType.DMA,
            pltpu.SemaphoreType.DMA,
        ],
    )
    def gather(table_hbm, idx_hbm, out_hbm, idx_v, rows_v, gsem, wsem0, wsem1):
        wid = lax.axis_index("s") * info.num_cores + lax.axis_index("c")
        base = wid * b_per_w
        wsems = (wsem0, wsem1)
        pltpu.sync_copy(idx_hbm.at[pl.ds(base, b_per_w)], idx_v)
        writebacks = [None, None]
        for c in range(n_chunks):
            sl = c % 2
            g = pltpu.make_async_copy(
                table_hbm.at[idx_v.at[pl.ds(c * chunk, chunk)]],
                rows_v.at[sl], gsem)
            g.start()
            if writebacks[sl] is not None:
                writebacks[sl].wait()
            g.wait()
            w = pltpu.make_async_copy(
                rows_v.at[sl],
                out_hbm.at[pl.ds(base + c * chunk, chunk)],
                wsems[sl])
            w.start()
            writebacks[sl] = w
        writebacks[0].wait()
        if writebacks[1] is not None:
            writebacks[1].wait()

    return gather


def kernel(x, codebook):
    b, t, d = x.shape
    m = b * t
    x_flat = x.reshape(m, d)
    c2 = (codebook ** 2).sum(axis=1)
    idx = _nearest_idx(x_flat, codebook, c2, block_rows=1024)
    out = _make_gather(d, m)(codebook, idx)
    return out.reshape(b, t, d)


# speedup vs baseline: 1.7596x; 1.0173x over previous
import functools

import jax
import jax.numpy as jnp
from jax import lax
from jax.experimental import pallas as pl
from jax.experimental.pallas import tpu as pltpu
from jax.experimental.pallas import tpu_sc as plsc


def _argmin_body(x_ref, cb_ref, c2_ref, idx_ref):
    n = cb_ref.shape[0]
    cb = cb_ref[...]
    c2 = c2_ref[...]
    r = x_ref.shape[0]
    sub = 256
    for s in range(r // sub):
        rows = pl.ds(s * sub, sub)
        x = x_ref[rows, :]
        xc = lax.dot_general(x, cb, (((1,), (1,)), ((), ())),
                             preferred_element_type=jnp.float32)
        x2 = jnp.sum(x * x, axis=1, keepdims=True)
        dists = (x2 + c2) - 2.0 * xc
        mn = jnp.min(dists, axis=1, keepdims=True)
        cand = jax.lax.broadcasted_iota(jnp.int32, dists.shape, 1)
        idx = jnp.min(jnp.where(dists == mn, cand, jnp.int32(n)), axis=1)
        idx_ref[0, 0, rows] = idx


def _nearest_idx(x_flat, cb, c2, block_rows):
    m, d = x_flat.shape
    n = cb.shape[0]
    n_blocks = m // block_rows
    out = pl.pallas_call(
        _argmin_body,
        grid=(n_blocks,),
        in_specs=[
            pl.BlockSpec((block_rows, d), lambda i: (i, 0)),
            pl.BlockSpec((n, d), lambda i: (0, 0)),
            pl.BlockSpec((1, n), lambda i: (0, 0)),
        ],
        out_specs=pl.BlockSpec((1, 1, block_rows), lambda i: (i, 0, 0)),
        out_shape=jax.ShapeDtypeStruct((n_blocks, 1, block_rows), jnp.int32),
    )(x_flat, cb, c2.reshape(1, n))
    return out.reshape(m)


def _make_gather(d, b_total):
    info = plsc.get_sparse_core_info()
    nw = info.num_cores * info.num_subcores
    b_per_w = b_total // nw
    chunk = min(32, b_per_w)
    n_chunks = b_per_w // chunk
    nbuf = min(3, n_chunks)
    mesh = plsc.VectorSubcoreMesh(core_axis_name="c", subcore_axis_name="s")

    @functools.partial(
        pl.kernel,
        mesh=mesh,
        out_type=jax.ShapeDtypeStruct((b_total, d), jnp.float32),
        scratch_types=[
            pltpu.VMEM((b_per_w,), jnp.int32),
            pltpu.VMEM((nbuf, chunk, d), jnp.float32),
            pltpu.SemaphoreType.DMA,
        ] + [pltpu.SemaphoreType.DMA] * nbuf,
    )
    def gather(table_hbm, idx_hbm, out_hbm, idx_v, rows_v, gsem, *wsems):
        wid = lax.axis_index("s") * info.num_cores + lax.axis_index("c")
        base = wid * b_per_w
        pltpu.sync_copy(idx_hbm.at[pl.ds(base, b_per_w)], idx_v)
        writebacks = [None] * nbuf
        for c in range(n_chunks):
            sl = c % nbuf
            g = pltpu.make_async_copy(
                table_hbm.at[idx_v.at[pl.ds(c * chunk, chunk)]],
                rows_v.at[sl], gsem)
            g.start()
            if writebacks[sl] is not None:
                writebacks[sl].wait()
            g.wait()
            w = pltpu.make_async_copy(
                rows_v.at[sl],
                out_hbm.at[pl.ds(base + c * chunk, chunk)],
                wsems[sl])
            w.start()
            writebacks[sl] = w
        for w in writebacks:
            if w is not None:
                w.wait()

    return gather


def kernel(x, codebook):
    b, t, d = x.shape
    m = b * t
    x_flat = x.reshape(m, d)
    c2 = (codebook ** 2).sum(axis=1)
    idx = _nearest_idx(x_flat, codebook, c2, block_rows=2048)
    out = _make_gather(d, m)(codebook, idx)
    return out.reshape(b, t, d)
